# baseline (speedup 1.0000x reference)
import functools

import jax
import jax.numpy as jnp
from jax import lax
from jax.experimental import pallas as pl
from jax.experimental.pallas import tpu as pltpu
from jax.experimental.pallas import tpu_sc as plsc

_NUM_CLASSES = 1000
_HIDDEN = 1024
_BATCH = 16384

_info = plsc.get_sparse_core_info()
_NC, _NS, _L = _info.num_cores, _info.num_subcores, _info.num_lanes
_NW = _NC * _NS
_BPW = _BATCH // _NW
_CH = 32
_NBUF = 2
_NCHUNK = _BPW // _CH


_mesh = plsc.VectorSubcoreMesh(core_axis_name="c", subcore_axis_name="s")


@functools.partial(
    pl.kernel,
    mesh=_mesh,
    out_type=jax.ShapeDtypeStruct((_BATCH, _HIDDEN), jnp.float32),
    scratch_types=[
        pltpu.VMEM((_BPW,), jnp.int32),
        pltpu.VMEM((_BPW,), jnp.int32),
        pltpu.VMEM((_BPW,), jnp.int32),
        pltpu.VMEM((_NBUF, _CH, _HIDDEN), jnp.float32),
        pltpu.SemaphoreType.DMA,
        pltpu.SemaphoreType.DMA,
    ],
)
def _embed(lab_hbm, fd_hbm, table_hbm, out_hbm, lab_v, fd_v, idx_v, buf, gsem, osem):
    wid = lax.axis_index("s") * _NC + lax.axis_index("c")
    base = wid * _BPW
    pltpu.async_copy(lab_hbm.at[pl.ds(base, _BPW)], lab_v, gsem).wait()
    pltpu.async_copy(fd_hbm.at[pl.ds(base, _BPW)], fd_v, osem).wait()

    for i in range(_BPW // _L):
        sl = pl.ds(i * _L, _L)
        idx_v[sl] = jnp.where(fd_v[sl] == 1, _NUM_CLASSES, lab_v[sl])

    def gather(c):
        return pltpu.async_copy(
            table_hbm.at[idx_v.at[pl.ds(c * _CH, _CH)]], buf.at[c % _NBUF], gsem
        )

    gh = [None] * _NCHUNK
    wh = [None] * _NCHUNK
    gh[0] = gather(0)
    for c in range(_NCHUNK):
        nxt = c + 1
        if nxt < _NCHUNK:
            if nxt - _NBUF >= 0:
                wh[nxt - _NBUF].wait()
            gh[nxt] = gather(nxt)
        gh[c].wait()
        wh[c] = pltpu.async_copy(
            buf.at[c % _NBUF], out_hbm.at[pl.ds(base + c * _CH, _CH)], osem
        )
    for c in range(_NCHUNK - _NBUF + 1, _NCHUNK):
        wh[c].wait()


def kernel(labels, force_drop_ids, embedding_table):
    return _embed(
        labels.astype(jnp.int32),
        force_drop_ids.astype(jnp.int32),
        embedding_table,
    )


# speedup vs baseline: 3.3534x; 3.3534x over previous
import functools

import jax
import jax.numpy as jnp
from jax import lax
from jax.experimental import pallas as pl
from jax.experimental.pallas import tpu as pltpu
from jax.experimental.pallas import tpu_sc as plsc

_NUM_CLASSES = 1000
_HIDDEN = 1024
_BATCH = 16384

_info = plsc.get_sparse_core_info()
_NC, _NS, _L = _info.num_cores, _info.num_subcores, _info.num_lanes
_NW = _NC * _NS
_BPW = _BATCH // _NW
_DEPTH = 16


_mesh = plsc.VectorSubcoreMesh(core_axis_name="c", subcore_axis_name="s")


@functools.partial(
    pl.kernel,
    mesh=_mesh,
    compiler_params=pltpu.CompilerParams(use_tc_tiling_on_sc=False),
    out_type=jax.ShapeDtypeStruct((_BATCH, _HIDDEN), jnp.float32),
    scratch_types=[
        pltpu.VMEM((_BPW,), jnp.int32),
        pltpu.VMEM((_BPW,), jnp.int32),
        pltpu.VMEM((_BPW,), jnp.int32),
        pltpu.VMEM_SHARED((_NUM_CLASSES + 1, _HIDDEN), jnp.float32),
        pltpu.SemaphoreType.DMA,
        pltpu.SemaphoreType.DMA,
    ],
)
def _embed(lab_hbm, fd_hbm, table_hbm, out_hbm, lab_v, fd_v, idx_v, table_sp, sem, osem):
    sid = lax.axis_index("s")
    wid = sid * _NC + lax.axis_index("c")
    base = wid * _BPW
    _ROWS_PER_TILE = (_NUM_CLASSES + 1 + _NS - 1) // _NS
    for t in range(_NS):
        start = t * _ROWS_PER_TILE
        cnt = min(_ROWS_PER_TILE, _NUM_CLASSES + 1 - start)
        @pl.when(sid == t)
        def _(start=start, cnt=cnt):
            pltpu.sync_copy(
                table_hbm.at[pl.ds(start, cnt)], table_sp.at[pl.ds(start, cnt)]
            )

    pltpu.async_copy(lab_hbm.at[pl.ds(base, _BPW)], lab_v, sem).wait()
    pltpu.async_copy(fd_hbm.at[pl.ds(base, _BPW)], fd_v, sem).wait()

    for i in range(_BPW // _L):
        sl = pl.ds(i * _L, _L)
        idx_v[sl] = jnp.where(fd_v[sl] == 1, _NUM_CLASSES, lab_v[sl])

    plsc.subcore_barrier()

    def row_copy(r, i):
        return pltpu.async_copy(
            table_sp.at[pl.ds(i, 1)], out_hbm.at[pl.ds(base + r, 1)], osem
        )

    handles = [None] * _BPW
    for g in range(_BPW // _L):
        vec = idx_v[pl.ds(g * _L, _L)]
        for j in range(_L):
            r = g * _L + j
            handles[r] = row_copy(r, vec[j])
            if r >= _DEPTH:
                handles[r - _DEPTH].wait()
    for r in range(_BPW - _DEPTH, _BPW):
        handles[r].wait()


def kernel(labels, force_drop_ids, embedding_table):
    return _embed(
        labels.astype(jnp.int32),
        force_drop_ids.astype(jnp.int32),
        embedding_table,
    )


# speedup vs baseline: 3.3573x; 1.0012x over previous
import functools

import jax
import jax.numpy as jnp
from jax import lax
from jax.experimental import pallas as pl
from jax.experimental.pallas import tpu as pltpu
from jax.experimental.pallas import tpu_sc as plsc

_NUM_CLASSES = 1000
_HIDDEN = 1024
_BATCH = 16384

_info = plsc.get_sparse_core_info()
_NC, _NS, _L = _info.num_cores, _info.num_subcores, _info.num_lanes
_NW = _NC * _NS
_BPW = _BATCH // _NW
_DEPTH = 64


_mesh = plsc.VectorSubcoreMesh(core_axis_name="c", subcore_axis_name="s")


@functools.partial(
    pl.kernel,
    mesh=_mesh,
    compiler_params=pltpu.CompilerParams(use_tc_tiling_on_sc=False),
    out_type=jax.ShapeDtypeStruct((_BATCH, _HIDDEN), jnp.float32),
    scratch_types=[
        pltpu.VMEM((_BPW,), jnp.int32),
        pltpu.VMEM((_BPW,), jnp.int32),
        pltpu.VMEM((_BPW,), jnp.int32),
        pltpu.VMEM_SHARED((_NUM_CLASSES + 1, _HIDDEN), jnp.float32),
        pltpu.SemaphoreType.DMA,
        pltpu.SemaphoreType.DMA,
    ],
)
def _embed(lab_hbm, fd_hbm, table_hbm, out_hbm, lab_v, fd_v, idx_v, table_sp, sem, osem):
    sid = lax.axis_index("s")
    wid = sid * _NC + lax.axis_index("c")
    base = wid * _BPW
    _ROWS_PER_TILE = (_NUM_CLASSES + 1 + _NS - 1) // _NS
    for t in range(_NS):
        start = t * _ROWS_PER_TILE
        cnt = min(_ROWS_PER_TILE, _NUM_CLASSES + 1 - start)
        @pl.when(sid == t)
        def _(start=start, cnt=cnt):
            pltpu.sync_copy(
                table_hbm.at[pl.ds(start, cnt)], table_sp.at[pl.ds(start, cnt)]
            )

    pltpu.async_copy(lab_hbm.at[pl.ds(base, _BPW)], lab_v, sem).wait()
    pltpu.async_copy(fd_hbm.at[pl.ds(base, _BPW)], fd_v, sem).wait()

    for i in range(_BPW // _L):
        sl = pl.ds(i * _L, _L)
        idx_v[sl] = jnp.where(fd_v[sl] == 1, _NUM_CLASSES, lab_v[sl])

    plsc.subcore_barrier()

    def row_copy(r, i):
        return pltpu.async_copy(
            table_sp.at[pl.ds(i, 1)], out_hbm.at[pl.ds(base + r, 1)], osem
        )

    handles = [None] * _BPW
    for g in range(_BPW // _L):
        vec = idx_v[pl.ds(g * _L, _L)]
        for j in range(_L):
            r = g * _L + j
            handles[r] = row_copy(r, vec[j])
            if r >= _DEPTH:
                handles[r - _DEPTH].wait()
    for r in range(_BPW - _DEPTH, _BPW):
        handles[r].wait()


def kernel(labels, force_drop_ids, embedding_table):
    return _embed(
        labels.astype(jnp.int32),
        force_drop_ids.astype(jnp.int32),
        embedding_table,
    )


# speedup vs baseline: 3.6084x; 1.0748x over previous
import functools

import jax
import jax.numpy as jnp
from jax import lax
from jax.experimental import pallas as pl
from jax.experimental.pallas import tpu as pltpu
from jax.experimental.pallas import tpu_sc as plsc

_NUM_CLASSES = 1000
_HIDDEN = 1024
_BATCH = 16384

_info = plsc.get_sparse_core_info()
_NC, _NS, _L = _info.num_cores, _info.num_subcores, _info.num_lanes
_NW = _NC * _NS
_BPW = _BATCH // _NW
_CH = 16
_NBUF = 3
_NCHUNK = _BPW // _CH


_mesh = plsc.VectorSubcoreMesh(core_axis_name="c", subcore_axis_name="s")


@functools.partial(
    pl.kernel,
    mesh=_mesh,
    compiler_params=pltpu.CompilerParams(use_tc_tiling_on_sc=False),
    out_type=jax.ShapeDtypeStruct((_BATCH, _HIDDEN), jnp.float32),
    scratch_types=[
        pltpu.VMEM((_BPW,), jnp.int32),
        pltpu.VMEM((_BPW,), jnp.int32),
        pltpu.VMEM((_BPW,), jnp.int32),
        pltpu.VMEM((_NBUF, _CH, _HIDDEN), jnp.float32),
        pltpu.VMEM_SHARED((_NUM_CLASSES + 1, _HIDDEN), jnp.float32),
        pltpu.SemaphoreType.DMA,
        pltpu.SemaphoreType.DMA,
    ],
)
def _embed(lab_hbm, fd_hbm, table_hbm, out_hbm, lab_v, fd_v, idx_v, buf, table_sp, sem, osem):
    sid = lax.axis_index("s")
    wid = sid * _NC + lax.axis_index("c")
    base = wid * _BPW
    _ROWS_PER_TILE = (_NUM_CLASSES + 1 + _NS - 1) // _NS
    for t in range(_NS):
        start = t * _ROWS_PER_TILE
        cnt = min(_ROWS_PER_TILE, _NUM_CLASSES + 1 - start)
        @pl.when(sid == t)
        def _(start=start, cnt=cnt):
            pltpu.sync_copy(
                table_hbm.at[pl.ds(start, cnt)], table_sp.at[pl.ds(start, cnt)]
            )

    pltpu.async_copy(lab_hbm.at[pl.ds(base, _BPW)], lab_v, sem).wait()
    pltpu.async_copy(fd_hbm.at[pl.ds(base, _BPW)], fd_v, sem).wait()

    for i in range(_BPW // _L):
        sl = pl.ds(i * _L, _L)
        idx_v[sl] = jnp.where(fd_v[sl] == 1, _NUM_CLASSES, lab_v[sl])

    plsc.subcore_barrier()

    wh = [None] * _NCHUNK
    rh = [[None] * _CH for _ in range(_NCHUNK)]

    def pull_chunk(c):
        s = c % _NBUF
        vec = idx_v[pl.ds(c * _CH, _CH)]
        for j in range(_CH):
            rh[c][j] = pltpu.async_copy(
                table_sp.at[pl.ds(vec[j], 1)], buf.at[s].at[pl.ds(j, 1)], sem
            )

    for c in range(_NCHUNK):
        if c < _NBUF - 1:
            pull_chunk(c)
    for c in range(_NCHUNK):
        nxt = c + _NBUF - 1
        if nxt < _NCHUNK:
            if nxt >= _NBUF:
                wh[nxt - _NBUF].wait()
            pull_chunk(nxt)
        for j in range(_CH):
            rh[c][j].wait()
        wh[c] = pltpu.async_copy(
            buf.at[c % _NBUF], out_hbm.at[pl.ds(base + c * _CH, _CH)], osem
        )
    for c in range(_NCHUNK - _NBUF, _NCHUNK):
        wh[c].wait()


def kernel(labels, force_drop_ids, embedding_table):
    return _embed(
        labels.astype(jnp.int32),
        force_drop_ids.astype(jnp.int32),
        embedding_table,
    )


# speedup vs baseline: 3.7558x; 1.0409x over previous
import functools

import jax
import jax.numpy as jnp
from jax import lax
from jax.experimental import pallas as pl
from jax.experimental.pallas import tpu as pltpu
from jax.experimental.pallas import tpu_sc as plsc

_NUM_CLASSES = 1000
_HIDDEN = 1024
_BATCH = 16384

_info = plsc.get_sparse_core_info()
_NC, _NS, _L = _info.num_cores, _info.num_subcores, _info.num_lanes
_NW = _NC * _NS
_BPW = _BATCH // _NW
_CH = 16
_NBUF = 3
_NCHUNK = _BPW // _CH


_mesh = plsc.VectorSubcoreMesh(core_axis_name="c", subcore_axis_name="s")


@functools.partial(
    pl.kernel,
    mesh=_mesh,
    compiler_params=pltpu.CompilerParams(use_tc_tiling_on_sc=False),
    out_type=jax.ShapeDtypeStruct((_BATCH, _HIDDEN), jnp.float32),
    scratch_types=[
        pltpu.VMEM((_BPW,), jnp.int32),
        pltpu.VMEM((_BPW,), jnp.int32),
        pltpu.VMEM((_BPW,), jnp.int32),
        pltpu.VMEM((_NBUF, _CH, _HIDDEN), jnp.float32),
        pltpu.VMEM((_CH,), jnp.int32),
        pltpu.VMEM((_CH,), jnp.int32),
        pltpu.VMEM((_CH,), jnp.int32),
        pltpu.VMEM_SHARED((_NUM_CLASSES + 1, _HIDDEN), jnp.float32),
        pltpu.SemaphoreType.DMA,
        pltpu.SemaphoreType.DMA,
    ],
)
def _embed(lab_hbm, fd_hbm, table_hbm, out_hbm, lab_v, fd_v, idx_v, buf, ix0, ix1, ix2, table_sp, sem, osem):
    sid = lax.axis_index("s")
    wid = sid * _NC + lax.axis_index("c")
    base = wid * _BPW
    _ROWS_PER_TILE = (_NUM_CLASSES + 1 + _NS - 1) // _NS
    for t in range(_NS):
        start = t * _ROWS_PER_TILE
        cnt = min(_ROWS_PER_TILE, _NUM_CLASSES + 1 - start)
        @pl.when(sid == t)
        def _(start=start, cnt=cnt):
            pltpu.sync_copy(
                table_hbm.at[pl.ds(start, cnt)], table_sp.at[pl.ds(start, cnt)]
            )

    pltpu.async_copy(lab_hbm.at[pl.ds(base, _BPW)], lab_v, sem).wait()
    pltpu.async_copy(fd_hbm.at[pl.ds(base, _BPW)], fd_v, sem).wait()

    for i in range(_BPW // _L):
        sl = pl.ds(i * _L, _L)
        idx_v[sl] = jnp.where(fd_v[sl] == 1, _NUM_CLASSES, lab_v[sl])

    plsc.subcore_barrier()

    wh = [None] * _NCHUNK
    rh = [[None] * _CH for _ in range(_NCHUNK)]

    ix = [ix0, ix1, ix2]

    def pull_chunk(c):
        s = c % _NBUF
        ix[s][pl.ds(0, _CH)] = idx_v[pl.ds(c * _CH, _CH)]
        rh[c][0] = pltpu.async_copy(table_sp.at[ix[s]], buf.at[s], sem)

    for c in range(_NCHUNK):
        if c < _NBUF - 1:
            pull_chunk(c)
    for c in range(_NCHUNK):
        nxt = c + _NBUF - 1
        if nxt < _NCHUNK:
            if nxt >= _NBUF:
                wh[nxt - _NBUF].wait()
            pull_chunk(nxt)
        rh[c][0].wait()
        wh[c] = pltpu.async_copy(
            buf.at[c % _NBUF], out_hbm.at[pl.ds(base + c * _CH, _CH)], osem
        )
    for c in range(_NCHUNK - _NBUF, _NCHUNK):
        wh[c].wait()


def kernel(labels, force_drop_ids, embedding_table):
    return _embed(
        labels.astype(jnp.int32),
        force_drop_ids.astype(jnp.int32),
        embedding_table,
    )
